# Initial kernel scaffold; baseline (speedup 1.0000x reference)
import jax
import jax.numpy as jnp
from jax.experimental import pallas as pl


def kernel(x, params):
    raise NotImplementedError("write your pallas kernel here")


# speedup vs baseline: 19.9892x; 19.9892x over previous
import functools

import jax
import jax.numpy as jnp
from jax import lax
from jax.experimental import pallas as pl
from jax.experimental.pallas import tpu as pltpu
from jax.experimental.pallas import tpu_sc as plsc

_B, _N, _K, _C, _H, _NL, _FFN = 4, 512, 20, 64, 8, 4, 128
_INV_SQRT_DH = 1.0 / 2.8284271247461903
_BN_DEN = 1.0000049999875


def _graph_kernel(x_ref, msp_ref, sidx_ref, r_ref, dist_ref):
    f32 = jnp.float32
    row_i = lax.broadcasted_iota(jnp.int32, (_N, _N), 0)
    col_i = lax.broadcasted_iota(jnp.int32, (_N, _N), 1)
    eyef = (row_i == col_i).astype(f32)

    def _tocol(v):
        return lax.dot_general(eyef, v, (((1,), (1,)), ((), ())),
                               preferred_element_type=f32)

    xg = x_ref[0]
    ip = lax.dot_general(xg, xg, (((0,), (0,)), ((), ())),
                         preferred_element_type=f32)
    xx = jnp.sum(xg * xg, axis=0, keepdims=True)
    inner = -2.0 * ip
    pd = (-xx) - inner - _tocol(xx)

    kbits = lax.bitcast_convert_type(pd, jnp.int32)
    key = kbits ^ ((kbits >> 31) & jnp.int32(0x7FFFFFFF))

    kk = jnp.int32(_K)
    cntpos = jnp.sum((key >= 0).astype(jnp.int32), axis=1, keepdims=True)
    base = jnp.where(cntpos >= kk, jnp.int32(0), jnp.int32(-2147483648))
    for b in range(30, -1, -1):
        cand = base + jnp.int32(1 << b)
        cnt = jnp.sum((key >= cand).astype(jnp.int32), axis=1, keepdims=True)
        base = jnp.where(cnt >= kk, cand, base)
    thr = base

    gt = key > thr
    eq = key == thr
    need = (kk - jnp.sum(gt.astype(jnp.int32), axis=1, keepdims=True)).astype(f32)
    tri = (row_i <= col_i).astype(f32)
    eqf = eq.astype(f32)
    rank = lax.dot_general(eqf, tri, (((1,), (0,)), ((), ())),
                           preferred_element_type=f32)
    adir = jnp.where(gt | (eq & (rank <= need)), 1.0, 0.0).astype(f32)

    adirt = lax.dot_general(adir, eyef, (((0,), (0,)), ((), ())),
                            preferred_element_type=f32)

    deg = jnp.sum(adir, axis=0, keepdims=True) + 1.0
    dinv = jnp.where(deg > 0.0, lax.rsqrt(deg), 0.0)
    msp_ref[0] = (_tocol(dinv) * (adirt + eyef)) * dinv

    asym = jnp.maximum(adir, adirt)
    afb = jnp.where((asym + eyef) > 0.0, 1.0, 0.0).astype(jnp.bfloat16)
    r_ref[...] = eyef.astype(jnp.bfloat16)
    dist_ref[...] = jnp.where(eyef > 0.0, 0.0, 255.0)

    def cond(carry):
        t, changed = carry
        return jnp.logical_and(t < 255, changed > 0)

    def body(carry):
        t, _ = carry
        r = r_ref[...]
        p = lax.dot_general(r, afb, (((1,), (0,)), ((), ())),
                            preferred_element_type=f32)
        rn = p > 0.0
        newm = jnp.logical_and(rn, jnp.logical_not(r > 0))
        dist_ref[...] = jnp.where(newm, (t + 1).astype(f32), dist_ref[...])
        r_ref[...] = rn.astype(jnp.bfloat16)
        return t + 1, jnp.sum(newm.astype(jnp.int32))

    lax.while_loop(cond, body, (jnp.int32(0), jnp.int32(1)))
    sidx_ref[0] = dist_ref[...].astype(jnp.int32)


def _graph_build(x):
    return pl.pallas_call(
        _graph_kernel,
        grid=(_B,),
        in_specs=[pl.BlockSpec((1, 3, _N), lambda b: (b, 0, 0))],
        out_specs=[pl.BlockSpec((1, _N, _N), lambda b: (b, 0, 0)),
                   pl.BlockSpec((1, _N, _N), lambda b: (b, 0, 0))],
        out_shape=[jax.ShapeDtypeStruct((_B, _N, _N), jnp.float32),
                   jax.ShapeDtypeStruct((_B, _N, _N), jnp.int32)],
        scratch_shapes=[pltpu.VMEM((_N, _N), jnp.bfloat16),
                        pltpu.VMEM((_N, _N), jnp.float32)],
    )(x)


def _bias_gather(sidx_flat, table):
    info = plsc.get_sparse_core_info()
    nw = info.num_cores * info.num_subcores
    tot = _B * _N * _N
    ch = tot // nw
    mesh = plsc.VectorSubcoreMesh(core_axis_name="c", subcore_axis_name="s")

    @functools.partial(
        pl.kernel, mesh=mesh,
        compiler_params=pltpu.CompilerParams(needs_layout_passes=False),
        out_type=jax.ShapeDtypeStruct((_B * _H * _N * _N,), jnp.float32),
        scratch_types=[pltpu.VMEM((ch,), jnp.int32),
                       pltpu.VMEM((ch,), jnp.float32),
                       pltpu.VMEM((256 * _H,), jnp.float32)],
    )
    def k(sidx_hbm, table_hbm, out_hbm, idx_v, buf_v, tab_v):
        wid = lax.axis_index("s") * info.num_cores + lax.axis_index("c")
        base = wid * ch
        pltpu.sync_copy(table_hbm, tab_v)
        pltpu.sync_copy(sidx_hbm.at[pl.ds(base, ch)], idx_v)
        bnum = base // (_N * _N)
        nm0 = base - bnum * (_N * _N)
        for h in range(_H):
            hvec = jnp.full((16,), h, jnp.int32)

            def body(g, _):
                idxv = idx_v[pl.ds(g * 16, 16)] * _H + hvec
                buf_v[pl.ds(g * 16, 16)] = plsc.load_gather(tab_v, [idxv])
                return 0

            lax.fori_loop(0, ch // 16, body, 0)
            off = bnum * (_H * _N * _N) + h * (_N * _N) + nm0
            pltpu.sync_copy(buf_v, out_hbm.at[pl.ds(off, ch)])

    return k(sidx_flat, table)


def _ln(h, g, b):
    m = jnp.mean(h, axis=1, keepdims=True)
    v = jnp.mean((h - m) * (h - m), axis=1, keepdims=True)
    return (h - m) / jnp.sqrt(v + 1e-5) * g + b


def _bn(h, g, b):
    return h / _BN_DEN * g + b


def _lrelu(h):
    return jnp.where(h >= 0.0, h, 0.2 * h)


def _mm(a, b):
    return lax.dot_general(a, b, (((1,), (0,)), ((), ())),
                           preferred_element_type=jnp.float32)


def _layers_kernel(x_ref, msp_ref, bias_ref,
                   c1w_ref, c1b_ref, bn1g_ref, bn1b_ref,
                   gw_ref, gb_ref, bng_ref, bnb_ref,
                   wq_ref, bq_ref, wk_ref, bk_ref, wv_ref, bv_ref,
                   wo_ref, bo_ref, ln1g_ref, ln1b_ref,
                   f1w_ref, f1b_ref, f2w_ref, f2b_ref, ln2g_ref, ln2b_ref,
                   l1w_ref, bn6g_ref, bn6b_ref, l2w_ref, l2b_ref,
                   bn7g_ref, bn7b_ref, l3w_ref, l3b_ref,
                   out_ref):
    f32 = jnp.float32
    xg = x_ref[0]
    msp = msp_ref[0]

    hw = lax.dot_general(xg, c1w_ref[...], (((0,), (0,)), ((), ())),
                         preferred_element_type=f32)
    h = _mm(msp, hw) + c1b_ref[...]
    h = _lrelu(_bn(h, bn1g_ref[...], bn1b_ref[...]))

    dh = _C // _H
    for l in range(_NL):
        h = _mm(msp, _mm(h, gw_ref[l])) + gb_ref[l]
        h = _lrelu(_bn(h, bng_ref[l], bnb_ref[l]))
        x0 = h
        q = _mm(x0, wq_ref[l]) + bq_ref[l]
        k = _mm(x0, wk_ref[l]) + bk_ref[l]
        v = _mm(x0, wv_ref[l]) + bv_ref[l]
        outs = []
        for hd in range(_H):
            qh = q[:, hd * dh:(hd + 1) * dh]
            kh = k[:, hd * dh:(hd + 1) * dh]
            vh = v[:, hd * dh:(hd + 1) * dh]
            s = lax.dot_general(qh, kh, (((1,), (1,)), ((), ())),
                                preferred_element_type=f32)
            s = s * _INV_SQRT_DH + bias_ref[0, hd]
            m = jnp.max(s, axis=1, keepdims=True)
            e = jnp.exp(s - m)
            p = e / jnp.sum(e, axis=1, keepdims=True)
            outs.append(_mm(p, vh))
        o = jnp.concatenate(outs, axis=1)
        o = _mm(o, wo_ref[l]) + bo_ref[l]
        y = _ln(x0 + o, ln1g_ref[l], ln1b_ref[l])
        f = jnp.maximum(_mm(y, f1w_ref[l]) + f1b_ref[l], 0.0)
        f = _mm(f, f2w_ref[l]) + f2b_ref[l]
        y = _ln(y + f, ln2g_ref[l], ln2b_ref[l])
        h = h + y

    xs = jnp.sum(h, axis=0, keepdims=True)
    xm = xs / jnp.float32(_N)
    z = jnp.concatenate([xm, xs], axis=1)
    z = _lrelu(_bn(_mm(z, l1w_ref[...]), bn6g_ref[...], bn6b_ref[...]))
    z = _lrelu(_bn(_mm(z, l2w_ref[...]) + l2b_ref[...],
                   bn7g_ref[...], bn7b_ref[...]))
    z = _mm(z, l3w_ref[...]) + l3b_ref[...]
    out_ref[0] = z


def _run_layers(x, msp, bias, wts):
    full = lambda a: pl.BlockSpec(a.shape, lambda b: (0,) * a.ndim)
    w_specs = [full(a) for a in wts]
    out = pl.pallas_call(
        _layers_kernel,
        grid=(_B,),
        in_specs=[pl.BlockSpec((1, 3, _N), lambda b: (b, 0, 0)),
                  pl.BlockSpec((1, _N, _N), lambda b: (b, 0, 0)),
                  pl.BlockSpec((1, _H, _N, _N), lambda b: (b, 0, 0, 0))]
                 + w_specs,
        out_specs=pl.BlockSpec((1, 1, 40), lambda b: (b, 0, 0)),
        out_shape=jax.ShapeDtypeStruct((_B, 1, 40), jnp.float32),
    )(x, msp, bias, *wts)
    return out.reshape(_B, 40)


def kernel(x, params):
    p = params
    msp, sidx = _graph_build(x)
    bias = _bias_gather(sidx.reshape(-1),
                        p['edge_dis'].reshape(-1)).reshape(_B, _H, _N, _N)

    r2 = lambda a: a.reshape(1, -1)
    r3 = lambda key: jnp.stack([p['l%d_%s' % (l, key)] for l in range(_NL)])
    r3b = lambda key: jnp.stack(
        [p['l%d_%s' % (l, key)].reshape(1, -1) for l in range(_NL)])
    wts = [
        p['conv1_W'], r2(p['conv1_b']), r2(p['bn1_g']), r2(p['bn1_b']),
        r3('gcn_W'), r3b('gcn_b'), r3b('bn_g'), r3b('bn_b'),
        r3('Wq'), r3b('bq'), r3('Wk'), r3b('bk'), r3('Wv'), r3b('bv'),
        r3('Wo'), r3b('bo'), r3b('ln1_g'), r3b('ln1_b'),
        r3('fc1_W'), r3b('fc1_b'), r3('fc2_W'), r3b('fc2_b'),
        r3b('ln2_g'), r3b('ln2_b'),
        p['lin1_W'], r2(p['bn6_g']), r2(p['bn6_b']),
        p['lin2_W'], r2(p['lin2_b']), r2(p['bn7_g']), r2(p['bn7_b']),
        p['lin3_W'], r2(p['lin3_b']),
    ]
    return _run_layers(x, msp, bias, wts)


# speedup vs baseline: 21.0103x; 1.0511x over previous
import functools

import jax
import jax.numpy as jnp
from jax import lax
from jax.experimental import pallas as pl
from jax.experimental.pallas import tpu as pltpu
from jax.experimental.pallas import tpu_sc as plsc

_B, _N, _K, _C, _H, _NL, _FFN = 4, 512, 20, 64, 8, 4, 128
_INV_SQRT_DH = 1.0 / 2.8284271247461903
_BN_DEN = 1.0000049999875


def _graph_kernel(x_ref, msp_ref, sidx_ref, r_ref, dist_ref):
    f32 = jnp.float32
    row_i = lax.broadcasted_iota(jnp.int32, (_N, _N), 0)
    col_i = lax.broadcasted_iota(jnp.int32, (_N, _N), 1)
    eyef = (row_i == col_i).astype(f32)

    def _tocol(v):
        return lax.dot_general(eyef, v, (((1,), (1,)), ((), ())),
                               preferred_element_type=f32)

    xg = x_ref[0]
    ip = lax.dot_general(xg, xg, (((0,), (0,)), ((), ())),
                         preferred_element_type=f32)
    xx = jnp.sum(xg * xg, axis=0, keepdims=True)
    inner = -2.0 * ip
    pd = (-xx) - inner - _tocol(xx)

    kbits = lax.bitcast_convert_type(pd, jnp.int32)
    key = kbits ^ ((kbits >> 31) & jnp.int32(0x7FFFFFFF))

    kk = jnp.int32(_K)
    cntpos = jnp.sum((key >= 0).astype(jnp.int32), axis=1, keepdims=True)
    base = jnp.where(cntpos >= kk, jnp.int32(0), jnp.int32(-2147483648))
    for b in range(30, -1, -1):
        cand = base + jnp.int32(1 << b)
        cnt = jnp.sum((key >= cand).astype(jnp.int32), axis=1, keepdims=True)
        base = jnp.where(cnt >= kk, cand, base)
    thr = base

    gt = key > thr
    eq = key == thr
    need = (kk - jnp.sum(gt.astype(jnp.int32), axis=1, keepdims=True)).astype(f32)
    tri = (row_i <= col_i).astype(f32)
    eqf = eq.astype(f32)
    rank = lax.dot_general(eqf, tri, (((1,), (0,)), ((), ())),
                           preferred_element_type=f32)
    adir = jnp.where(gt | (eq & (rank <= need)), 1.0, 0.0).astype(f32)

    adirt = lax.dot_general(adir, eyef, (((0,), (0,)), ((), ())),
                            preferred_element_type=f32)

    deg = jnp.sum(adir, axis=0, keepdims=True) + 1.0
    dinv = jnp.where(deg > 0.0, lax.rsqrt(deg), 0.0)
    msp_ref[0] = (_tocol(dinv) * (adirt + eyef)) * dinv

    asym = jnp.maximum(adir, adirt)
    afb = jnp.where((asym + eyef) > 0.0, 1.0, 0.0).astype(jnp.bfloat16)
    ones_col = jnp.full((_N, 1), 1.0, jnp.bfloat16)
    r_ref[...] = eyef.astype(jnp.bfloat16)
    dist_ref[...] = jnp.zeros((_N, _N), f32)

    def cond(carry):
        t, prev, cur = carry
        return jnp.logical_and(t < 255, cur > prev)

    def body(carry):
        t, _, cur = carry
        r = r_ref[...]
        dist_ref[...] = dist_ref[...] + (1.0 - r.astype(f32))
        p = lax.dot_general(r, afb, (((1,), (0,)), ((), ())),
                            preferred_element_type=f32)
        rnb = (p > 0.0).astype(jnp.bfloat16)
        r_ref[...] = rnb
        s1 = lax.dot_general(rnb, ones_col, (((1,), (0,)), ((), ())),
                             preferred_element_type=f32)
        return t + 1, cur, jnp.sum(s1)

    lax.while_loop(cond, body, (jnp.int32(0), jnp.float32(0.0),
                                jnp.float32(_N)))
    sidx_ref[0] = jnp.where(r_ref[...] > 0, dist_ref[...], 255.0
                            ).astype(jnp.int32)


def _graph_build(x):
    return pl.pallas_call(
        _graph_kernel,
        grid=(_B,),
        in_specs=[pl.BlockSpec((1, 3, _N), lambda b: (b, 0, 0))],
        out_specs=[pl.BlockSpec((1, _N, _N), lambda b: (b, 0, 0)),
                   pl.BlockSpec((1, _N, _N), lambda b: (b, 0, 0))],
        out_shape=[jax.ShapeDtypeStruct((_B, _N, _N), jnp.float32),
                   jax.ShapeDtypeStruct((_B, _N, _N), jnp.int32)],
        scratch_shapes=[pltpu.VMEM((_N, _N), jnp.bfloat16),
                        pltpu.VMEM((_N, _N), jnp.float32)],
    )(x)


def _bias_gather(sidx_flat, tables):
    info = plsc.get_sparse_core_info()
    nw = info.num_cores * info.num_subcores
    tot = _B * _N * _N
    ch = tot // nw
    sub = 4096
    nsub = ch // sub
    mesh = plsc.VectorSubcoreMesh(core_axis_name="c", subcore_axis_name="s")

    @functools.partial(
        pl.kernel, mesh=mesh,
        compiler_params=pltpu.CompilerParams(needs_layout_passes=False),
        out_type=jax.ShapeDtypeStruct((_B * _H * _N * _N,), jnp.float32),
        scratch_types=[pltpu.VMEM((ch,), jnp.int32)]
                      + [pltpu.VMEM((sub,), jnp.float32) for _ in range(_H)]
                      + [pltpu.VMEM((256,), jnp.float32) for _ in range(_H)],
    )
    def k(sidx_hbm, tab_hbm, out_hbm, idx_v, *bufs_tabs):
        bufs, tabs = bufs_tabs[:_H], bufs_tabs[_H:]
        wid = lax.axis_index("s") * info.num_cores + lax.axis_index("c")
        base = wid * ch
        for h in range(_H):
            pltpu.sync_copy(tab_hbm.at[pl.ds(h * 256, 256)], tabs[h])
        pltpu.sync_copy(sidx_hbm.at[pl.ds(base, ch)], idx_v)
        bnum = base // (_N * _N)
        nm0 = base - bnum * (_N * _N)
        obase = bnum * (_H * _N * _N) + nm0
        for si in range(nsub):
            def body(g, _, si=si):
                idxv = idx_v[pl.ds(si * sub + g * 16, 16)]
                for h in range(_H):
                    bufs[h][pl.ds(g * 16, 16)] = plsc.load_gather(tabs[h],
                                                                  [idxv])
                return 0

            lax.fori_loop(0, sub // 16, body, 0)
            for h in range(_H):
                pltpu.sync_copy(
                    bufs[h],
                    out_hbm.at[pl.ds(obase + h * (_N * _N) + si * sub, sub)])

    return k(sidx_flat, tables)


def _ln(h, g, b):
    m = jnp.mean(h, axis=1, keepdims=True)
    v = jnp.mean((h - m) * (h - m), axis=1, keepdims=True)
    return (h - m) / jnp.sqrt(v + 1e-5) * g + b


def _bn(h, g, b):
    return h / _BN_DEN * g + b


def _lrelu(h):
    return jnp.where(h >= 0.0, h, 0.2 * h)


def _mm(a, b):
    return lax.dot_general(a, b, (((1,), (0,)), ((), ())),
                           preferred_element_type=jnp.float32)


def _layers_kernel(x_ref, msp_ref, bias_ref,
                   c1w_ref, c1b_ref, bn1g_ref, bn1b_ref,
                   gw_ref, gb_ref, bng_ref, bnb_ref,
                   wq_ref, bq_ref, wk_ref, bk_ref, wv_ref, bv_ref,
                   wo_ref, bo_ref, ln1g_ref, ln1b_ref,
                   f1w_ref, f1b_ref, f2w_ref, f2b_ref, ln2g_ref, ln2b_ref,
                   l1w_ref, bn6g_ref, bn6b_ref, l2w_ref, l2b_ref,
                   bn7g_ref, bn7b_ref, l3w_ref, l3b_ref,
                   out_ref):
    f32 = jnp.float32
    xg = x_ref[0]
    msp = msp_ref[0]

    hw = lax.dot_general(xg, c1w_ref[...], (((0,), (0,)), ((), ())),
                         preferred_element_type=f32)
    h = _mm(msp, hw) + c1b_ref[...]
    h = _lrelu(_bn(h, bn1g_ref[...], bn1b_ref[...]))

    dh = _C // _H
    for l in range(_NL):
        h = _mm(msp, _mm(h, gw_ref[l])) + gb_ref[l]
        h = _lrelu(_bn(h, bng_ref[l], bnb_ref[l]))
        x0 = h
        q = (_mm(x0, wq_ref[l]) + bq_ref[l]) * _INV_SQRT_DH
        k = _mm(x0, wk_ref[l]) + bk_ref[l]
        v = _mm(x0, wv_ref[l]) + bv_ref[l]
        outs = []
        for hd in range(_H):
            qh = q[:, hd * dh:(hd + 1) * dh]
            kh = k[:, hd * dh:(hd + 1) * dh]
            vh = v[:, hd * dh:(hd + 1) * dh]
            s = lax.dot_general(qh, kh, (((1,), (1,)), ((), ())),
                                preferred_element_type=f32)
            e = jnp.exp(s + bias_ref[0, hd])
            p = e * (1.0 / jnp.sum(e, axis=1, keepdims=True))
            outs.append(_mm(p, vh))
        o = jnp.concatenate(outs, axis=1)
        o = _mm(o, wo_ref[l]) + bo_ref[l]
        y = _ln(x0 + o, ln1g_ref[l], ln1b_ref[l])
        f = jnp.maximum(_mm(y, f1w_ref[l]) + f1b_ref[l], 0.0)
        f = _mm(f, f2w_ref[l]) + f2b_ref[l]
        y = _ln(y + f, ln2g_ref[l], ln2b_ref[l])
        h = h + y

    xs = jnp.sum(h, axis=0, keepdims=True)
    xm = xs / jnp.float32(_N)
    z = jnp.concatenate([xm, xs], axis=1)
    z = _lrelu(_bn(_mm(z, l1w_ref[...]), bn6g_ref[...], bn6b_ref[...]))
    z = _lrelu(_bn(_mm(z, l2w_ref[...]) + l2b_ref[...],
                   bn7g_ref[...], bn7b_ref[...]))
    z = _mm(z, l3w_ref[...]) + l3b_ref[...]
    out_ref[0] = z


def _run_layers(x, msp, bias, wts):
    full = lambda a: pl.BlockSpec(a.shape, lambda b: (0,) * a.ndim)
    w_specs = [full(a) for a in wts]
    out = pl.pallas_call(
        _layers_kernel,
        grid=(_B,),
        in_specs=[pl.BlockSpec((1, 3, _N), lambda b: (b, 0, 0)),
                  pl.BlockSpec((1, _N, _N), lambda b: (b, 0, 0)),
                  pl.BlockSpec((1, _H, _N, _N), lambda b: (b, 0, 0, 0))]
                 + w_specs,
        out_specs=pl.BlockSpec((1, 1, 40), lambda b: (b, 0, 0)),
        out_shape=jax.ShapeDtypeStruct((_B, 1, 40), jnp.float32),
    )(x, msp, bias, *wts)
    return out.reshape(_B, 40)


def kernel(x, params):
    p = params
    msp, sidx = _graph_build(x)
    bias = _bias_gather(sidx.reshape(-1),
                        p['edge_dis'].T.reshape(-1)).reshape(_B, _H, _N, _N)

    r2 = lambda a: a.reshape(1, -1)
    r3 = lambda key: jnp.stack([p['l%d_%s' % (l, key)] for l in range(_NL)])
    r3b = lambda key: jnp.stack(
        [p['l%d_%s' % (l, key)].reshape(1, -1) for l in range(_NL)])
    wts = [
        p['conv1_W'], r2(p['conv1_b']), r2(p['bn1_g']), r2(p['bn1_b']),
        r3('gcn_W'), r3b('gcn_b'), r3b('bn_g'), r3b('bn_b'),
        r3('Wq'), r3b('bq'), r3('Wk'), r3b('bk'), r3('Wv'), r3b('bv'),
        r3('Wo'), r3b('bo'), r3b('ln1_g'), r3b('ln1_b'),
        r3('fc1_W'), r3b('fc1_b'), r3('fc2_W'), r3b('fc2_b'),
        r3b('ln2_g'), r3b('ln2_b'),
        p['lin1_W'], r2(p['bn6_g']), r2(p['bn6_b']),
        p['lin2_W'], r2(p['lin2_b']), r2(p['bn7_g']), r2(p['bn7_b']),
        p['lin3_W'], r2(p['lin3_b']),
    ]
    return _run_layers(x, msp, bias, wts)


# speedup vs baseline: 22.5920x; 1.0753x over previous
import functools

import jax
import jax.numpy as jnp
from jax import lax
from jax.experimental import pallas as pl
from jax.experimental.pallas import tpu as pltpu
from jax.experimental.pallas import tpu_sc as plsc

_B, _N, _K, _C, _H, _NL, _FFN = 4, 512, 20, 64, 8, 4, 128
_INV_SQRT_DH = 1.0 / 2.8284271247461903
_BN_DEN = 1.0000049999875


def _graph_kernel(x_ref, msp_ref, sidx_ref, r_ref, dist_ref):
    f32 = jnp.float32
    row_i = lax.broadcasted_iota(jnp.int32, (_N, _N), 0)
    col_i = lax.broadcasted_iota(jnp.int32, (_N, _N), 1)
    eyef = (row_i == col_i).astype(f32)

    def _tocol(v):
        return lax.dot_general(eyef, v, (((1,), (1,)), ((), ())),
                               preferred_element_type=f32)

    xg = x_ref[0]
    ip = lax.dot_general(xg, xg, (((0,), (0,)), ((), ())),
                         preferred_element_type=f32)
    xx = jnp.sum(xg * xg, axis=0, keepdims=True)
    inner = -2.0 * ip
    pd = (-xx) - inner - _tocol(xx)

    kbits = lax.bitcast_convert_type(pd, jnp.int32)
    key = kbits ^ ((kbits >> 31) & jnp.int32(0x7FFFFFFF))

    kk = jnp.int32(_K)
    cntpos = jnp.sum((key >= 0).astype(jnp.int32), axis=1, keepdims=True)
    base = jnp.where(cntpos >= kk, jnp.int32(0), jnp.int32(-2147483648))
    for b in range(30, -1, -1):
        cand = base + jnp.int32(1 << b)
        cnt = jnp.sum((key >= cand).astype(jnp.int32), axis=1, keepdims=True)
        base = jnp.where(cnt >= kk, cand, base)
    thr = base

    gt = key > thr
    eq = key == thr
    need = (kk - jnp.sum(gt.astype(jnp.int32), axis=1, keepdims=True)).astype(f32)
    tri = (row_i <= col_i).astype(f32)
    eqf = eq.astype(f32)
    rank = lax.dot_general(eqf, tri, (((1,), (0,)), ((), ())),
                           preferred_element_type=f32)
    adir = jnp.where(gt | (eq & (rank <= need)), 1.0, 0.0).astype(f32)

    adirt = lax.dot_general(adir, eyef, (((0,), (0,)), ((), ())),
                            preferred_element_type=f32)

    deg = jnp.sum(adir, axis=0, keepdims=True) + 1.0
    dinv = jnp.where(deg > 0.0, lax.rsqrt(deg), 0.0)
    msp_ref[0] = (_tocol(dinv) * (adirt + eyef)) * dinv

    asym = jnp.maximum(adir, adirt)
    afb = jnp.where((asym + eyef) > 0.0, 1.0, 0.0).astype(jnp.bfloat16)
    ones_col = jnp.full((_N, 1), 1.0, jnp.bfloat16)
    r_ref[...] = eyef.astype(jnp.bfloat16)
    dist_ref[...] = jnp.zeros((_N, _N), f32)

    def cond(carry):
        t, prev, cur = carry
        return jnp.logical_and(t < 64, cur > prev)

    def body(carry):
        t, _, cur = carry
        rnb = r_ref[...]
        for _ in range(4):
            dist_ref[...] = dist_ref[...] + (1.0 - rnb.astype(f32))
            p = lax.dot_general(rnb, afb, (((1,), (0,)), ((), ())),
                                preferred_element_type=f32)
            rnb = (p > 0.0).astype(jnp.bfloat16)
        r_ref[...] = rnb
        s1 = lax.dot_general(rnb, ones_col, (((1,), (0,)), ((), ())),
                             preferred_element_type=f32)
        return t + 1, cur, jnp.sum(s1)

    lax.while_loop(cond, body, (jnp.int32(0), jnp.float32(0.0),
                                jnp.float32(_N)))
    sidx_ref[0] = jnp.where(r_ref[...] > 0,
                            jnp.minimum(dist_ref[...], 255.0),
                            255.0).astype(jnp.int32)


def _graph_build(x):
    return pl.pallas_call(
        _graph_kernel,
        grid=(_B,),
        in_specs=[pl.BlockSpec((1, 3, _N), lambda b: (b, 0, 0))],
        out_specs=[pl.BlockSpec((1, _N, _N), lambda b: (b, 0, 0)),
                   pl.BlockSpec((1, _N, _N), lambda b: (b, 0, 0))],
        out_shape=[jax.ShapeDtypeStruct((_B, _N, _N), jnp.float32),
                   jax.ShapeDtypeStruct((_B, _N, _N), jnp.int32)],
        scratch_shapes=[pltpu.VMEM((_N, _N), jnp.bfloat16),
                        pltpu.VMEM((_N, _N), jnp.float32)],
    )(x)


def _bias_gather(sidx_flat, tables):
    info = plsc.get_sparse_core_info()
    nw = info.num_cores * info.num_subcores
    tot = _B * _N * _N
    ch = tot // nw
    sub = 4096
    nsub = ch // sub
    mesh = plsc.VectorSubcoreMesh(core_axis_name="c", subcore_axis_name="s")

    @functools.partial(
        pl.kernel, mesh=mesh,
        compiler_params=pltpu.CompilerParams(needs_layout_passes=False),
        out_type=jax.ShapeDtypeStruct((_B, _H, _N * _N), jnp.float32),
        scratch_types=[pltpu.VMEM((ch,), jnp.int32),
                       pltpu.VMEM((_H, sub), jnp.float32),
                       pltpu.VMEM((_H, sub), jnp.float32),
                       pltpu.SemaphoreType.DMA,
                       pltpu.SemaphoreType.DMA]
                      + [pltpu.VMEM((256,), jnp.float32) for _ in range(_H)],
    )
    def k(sidx_hbm, tab_hbm, out_hbm, idx_v, buf0, buf1, sem0, sem1, *tabs):
        banks = (buf0, buf1)
        sems = (sem0, sem1)
        wid = lax.axis_index("s") * info.num_cores + lax.axis_index("c")
        base = wid * ch
        for h in range(_H):
            pltpu.sync_copy(tab_hbm.at[pl.ds(h * 256, 256)], tabs[h])
        pltpu.sync_copy(sidx_hbm.at[pl.ds(base, ch)], idx_v)
        bnum = base // (_N * _N)
        nm0 = base - bnum * (_N * _N)
        copies = [None, None]
        for si in range(nsub):
            buf = banks[si % 2]
            if copies[si % 2] is not None:
                copies[si % 2].wait()

            def body(g, _, si=si, buf=buf):
                idxv = idx_v[pl.ds(si * sub + g * 16, 16)]
                for h in range(_H):
                    buf[h, pl.ds(g * 16, 16)] = plsc.load_gather(tabs[h],
                                                                 [idxv])
                return 0

            lax.fori_loop(0, sub // 16, body, 0)
            cp = pltpu.make_async_copy(
                buf, out_hbm.at[bnum, :, pl.ds(nm0 + si * sub, sub)],
                sems[si % 2])
            cp.start()
            copies[si % 2] = cp
        copies[(nsub - 1) % 2].wait()
        copies[nsub % 2].wait()

    return k(sidx_flat, tables).reshape(_B * _H * _N * _N)


def _ln(h, g, b):
    m = jnp.mean(h, axis=1, keepdims=True)
    v = jnp.mean((h - m) * (h - m), axis=1, keepdims=True)
    return (h - m) / jnp.sqrt(v + 1e-5) * g + b


def _bn(h, g, b):
    return h / _BN_DEN * g + b


def _lrelu(h):
    return jnp.where(h >= 0.0, h, 0.2 * h)


def _mm(a, b):
    return lax.dot_general(a, b, (((1,), (0,)), ((), ())),
                           preferred_element_type=jnp.float32)


def _layers_kernel(x_ref, msp_ref, bias_ref,
                   c1w_ref, c1b_ref, bn1g_ref, bn1b_ref,
                   gw_ref, gb_ref, bng_ref, bnb_ref,
                   wq_ref, bq_ref, wk_ref, bk_ref, wv_ref, bv_ref,
                   wo_ref, bo_ref, ln1g_ref, ln1b_ref,
                   f1w_ref, f1b_ref, f2w_ref, f2b_ref, ln2g_ref, ln2b_ref,
                   l1w_ref, bn6g_ref, bn6b_ref, l2w_ref, l2b_ref,
                   bn7g_ref, bn7b_ref, l3w_ref, l3b_ref,
                   out_ref):
    f32 = jnp.float32
    xg = x_ref[0]
    msp = msp_ref[0]

    hw = lax.dot_general(xg, c1w_ref[...], (((0,), (0,)), ((), ())),
                         preferred_element_type=f32)
    h = _mm(msp, hw) + c1b_ref[...]
    h = _lrelu(_bn(h, bn1g_ref[...], bn1b_ref[...]))

    dh = _C // _H
    for l in range(_NL):
        h = _mm(msp, _mm(h, gw_ref[l])) + gb_ref[l]
        h = _lrelu(_bn(h, bng_ref[l], bnb_ref[l]))
        x0 = h
        q = (_mm(x0, wq_ref[l]) + bq_ref[l]) * _INV_SQRT_DH
        k = _mm(x0, wk_ref[l]) + bk_ref[l]
        v = _mm(x0, wv_ref[l]) + bv_ref[l]
        outs = []
        for hd in range(_H):
            qh = q[:, hd * dh:(hd + 1) * dh]
            kh = k[:, hd * dh:(hd + 1) * dh]
            vh = v[:, hd * dh:(hd + 1) * dh]
            s = lax.dot_general(qh, kh, (((1,), (1,)), ((), ())),
                                preferred_element_type=f32)
            e = jnp.exp(s + bias_ref[0, hd])
            p = e * (1.0 / jnp.sum(e, axis=1, keepdims=True))
            outs.append(_mm(p, vh))
        o = jnp.concatenate(outs, axis=1)
        o = _mm(o, wo_ref[l]) + bo_ref[l]
        y = _ln(x0 + o, ln1g_ref[l], ln1b_ref[l])
        f = jnp.maximum(_mm(y, f1w_ref[l]) + f1b_ref[l], 0.0)
        f = _mm(f, f2w_ref[l]) + f2b_ref[l]
        y = _ln(y + f, ln2g_ref[l], ln2b_ref[l])
        h = h + y

    xs = jnp.sum(h, axis=0, keepdims=True)
    xm = xs / jnp.float32(_N)
    z = jnp.concatenate([xm, xs], axis=1)
    z = _lrelu(_bn(_mm(z, l1w_ref[...]), bn6g_ref[...], bn6b_ref[...]))
    z = _lrelu(_bn(_mm(z, l2w_ref[...]) + l2b_ref[...],
                   bn7g_ref[...], bn7b_ref[...]))
    z = _mm(z, l3w_ref[...]) + l3b_ref[...]
    out_ref[0] = z


def _run_layers(x, msp, bias, wts):
    full = lambda a: pl.BlockSpec(a.shape, lambda b: (0,) * a.ndim)
    w_specs = [full(a) for a in wts]
    out = pl.pallas_call(
        _layers_kernel,
        grid=(_B,),
        in_specs=[pl.BlockSpec((1, 3, _N), lambda b: (b, 0, 0)),
                  pl.BlockSpec((1, _N, _N), lambda b: (b, 0, 0)),
                  pl.BlockSpec((1, _H, _N, _N), lambda b: (b, 0, 0, 0))]
                 + w_specs,
        out_specs=pl.BlockSpec((1, 1, 40), lambda b: (b, 0, 0)),
        out_shape=jax.ShapeDtypeStruct((_B, 1, 40), jnp.float32),
    )(x, msp, bias, *wts)
    return out.reshape(_B, 40)


def kernel(x, params):
    p = params
    msp, sidx = _graph_build(x)
    bias = _bias_gather(sidx.reshape(-1),
                        p['edge_dis'].T.reshape(-1)).reshape(_B, _H, _N, _N)

    r2 = lambda a: a.reshape(1, -1)
    r3 = lambda key: jnp.stack([p['l%d_%s' % (l, key)] for l in range(_NL)])
    r3b = lambda key: jnp.stack(
        [p['l%d_%s' % (l, key)].reshape(1, -1) for l in range(_NL)])
    wts = [
        p['conv1_W'], r2(p['conv1_b']), r2(p['bn1_g']), r2(p['bn1_b']),
        r3('gcn_W'), r3b('gcn_b'), r3b('bn_g'), r3b('bn_b'),
        r3('Wq'), r3b('bq'), r3('Wk'), r3b('bk'), r3('Wv'), r3b('bv'),
        r3('Wo'), r3b('bo'), r3b('ln1_g'), r3b('ln1_b'),
        r3('fc1_W'), r3b('fc1_b'), r3('fc2_W'), r3b('fc2_b'),
        r3b('ln2_g'), r3b('ln2_b'),
        p['lin1_W'], r2(p['bn6_g']), r2(p['bn6_b']),
        p['lin2_W'], r2(p['lin2_b']), r2(p['bn7_g']), r2(p['bn7_b']),
        p['lin3_W'], r2(p['lin3_b']),
    ]
    return _run_layers(x, msp, bias, wts)
